# baseline (speedup 1.0000x reference)
import functools

import jax
import jax.numpy as jnp
from jax import lax
from jax.experimental import pallas as pl
from jax.experimental.pallas import tpu as pltpu
from jax.experimental.pallas import tpu_sc as plsc

N = 10000
D = 128
E = 320000

NC = 2
NS = 16
NW = NC * NS

CH = 128
K = 80
G = K // 4
EW = K * CH
E_PAD = NW * EW
PAD = E_PAD - E
DEGW = 8

NROW = 10240
RPT = NROW // NS
DUMP = N
ZR = 80

R = 400
GRID = N // R


def _zero_acc(zb, acc, s, groups):
    def fill(i, _):
        zb[i // 8, pl.ds((i % 8) * 16, 16)] = jnp.zeros((16,), jnp.float32)
        return 0

    lax.fori_loop(0, ZR * 8, fill, 0)

    def zero(t, _):
        pltpu.sync_copy(zb, acc.at[pl.ds(s * RPT + t * ZR, ZR)])
        return 0

    lax.fori_loop(0, groups, zero, 0)


def _copy_out(acc, out_hbm, c, s):
    pltpu.sync_copy(
        acc.at[pl.ds(s * RPT, RPT)],
        out_hbm.at[pl.ds(c * NROW + s * RPT, RPT)],
    )


def _sc_deg_body(dst_hbm, out_hbm, dstv, ones, zb, acc, sem):
    c = lax.axis_index("c")
    s = lax.axis_index("s")
    wid = c * NS + s

    def fill(i, _):
        ones[i // 8, pl.ds((i % 8) * 16, 16)] = jnp.ones((16,), jnp.float32)
        return 0

    lax.fori_loop(0, CH * 8, fill, 0)
    _zero_acc(zb, acc, s, RPT // ZR)
    pltpu.sync_copy(dst_hbm.at[pl.ds(wid * K, K)], dstv)
    plsc.subcore_barrier()

    for j in range(DEGW):
        pltpu.async_copy(ones, acc.at[dstv.at[j]], sem, add=True)

    def chunk(j, _):
        pltpu.make_async_copy(ones, acc.at[dstv.at[j]], sem).wait()
        pltpu.async_copy(ones, acc.at[dstv.at[j + DEGW]], sem, add=True)
        return 0

    lax.fori_loop(0, K - DEGW, chunk, 0)
    for j in range(DEGW):
        pltpu.make_async_copy(ones, acc.at[dstv.at[0]], sem).wait()
    plsc.subcore_barrier()
    _copy_out(acc, out_hbm, c, s)


def _sc_agg_body(g_hbm, idx_hbm, out_hbm, idxA, idxB, rows0, rows1,
                 acc, gs0, gs1, ss0, ss1, semA, semB):
    c = lax.axis_index("c")
    s = lax.axis_index("s")
    wid = c * NS + s

    def fill(i, _):
        rows1[i // 8, pl.ds((i % 8) * 16, 16)] = jnp.zeros((16,), jnp.float32)
        return 0

    lax.fori_loop(0, CH * 8, fill, 0)

    def zero(t, _):
        pltpu.sync_copy(rows1, acc.at[pl.ds(s * RPT + t * CH, CH)])
        return 0

    lax.fori_loop(0, RPT // CH, zero, 0)

    pltpu.sync_copy(idx_hbm.at[pl.ds(wid * G * 8, 8)], idxA)
    plsc.subcore_barrier()

    pltpu.async_copy(rows1, acc.at[idxA.at[4]], ss1, add=True)
    pltpu.async_copy(g_hbm.at[idxA.at[0]], rows0, gs0)

    def two_groups(it, _):
        offB = (wid * G + 2 * it + 1) * 8
        offA = (wid * G + jnp.minimum(2 * it + 2, G - 1)) * 8

        pltpu.make_async_copy(g_hbm.at[idxA.at[0]], rows0, gs0).wait()
        pltpu.async_copy(rows0, acc.at[idxA.at[4]], ss0, add=True)
        pltpu.make_async_copy(rows1, acc.at[idxA.at[5]], ss1).wait()
        pltpu.async_copy(idx_hbm.at[pl.ds(offB, 8)], idxB, semB)
        pltpu.async_copy(g_hbm.at[idxA.at[1]], rows1, gs1)

        pltpu.make_async_copy(g_hbm.at[idxA.at[1]], rows1, gs1).wait()
        pltpu.async_copy(rows1, acc.at[idxA.at[5]], ss1, add=True)
        pltpu.make_async_copy(rows0, acc.at[idxA.at[4]], ss0).wait()
        pltpu.async_copy(g_hbm.at[idxA.at[2]], rows0, gs0)

        pltpu.make_async_copy(g_hbm.at[idxA.at[2]], rows0, gs0).wait()
        pltpu.async_copy(rows0, acc.at[idxA.at[6]], ss0, add=True)
        pltpu.make_async_copy(rows1, acc.at[idxA.at[5]], ss1).wait()
        pltpu.async_copy(g_hbm.at[idxA.at[3]], rows1, gs1)

        pltpu.make_async_copy(g_hbm.at[idxA.at[3]], rows1, gs1).wait()
        pltpu.async_copy(rows1, acc.at[idxA.at[7]], ss1, add=True)
        pltpu.make_async_copy(rows0, acc.at[idxA.at[6]], ss0).wait()
        pltpu.make_async_copy(idx_hbm.at[pl.ds(offB, 8)], idxB, semB).wait()
        pltpu.async_copy(g_hbm.at[idxB.at[0]], rows0, gs0)

        pltpu.make_async_copy(g_hbm.at[idxB.at[0]], rows0, gs0).wait()
        pltpu.async_copy(rows0, acc.at[idxB.at[4]], ss0, add=True)
        pltpu.make_async_copy(rows1, acc.at[idxB.at[5]], ss1).wait()
        pltpu.async_copy(idx_hbm.at[pl.ds(offA, 8)], idxA, semA)
        pltpu.async_copy(g_hbm.at[idxB.at[1]], rows1, gs1)

        pltpu.make_async_copy(g_hbm.at[idxB.at[1]], rows1, gs1).wait()
        pltpu.async_copy(rows1, acc.at[idxB.at[5]], ss1, add=True)
        pltpu.make_async_copy(rows0, acc.at[idxB.at[4]], ss0).wait()
        pltpu.async_copy(g_hbm.at[idxB.at[2]], rows0, gs0)

        pltpu.make_async_copy(g_hbm.at[idxB.at[2]], rows0, gs0).wait()
        pltpu.async_copy(rows0, acc.at[idxB.at[6]], ss0, add=True)
        pltpu.make_async_copy(rows1, acc.at[idxB.at[5]], ss1).wait()
        pltpu.async_copy(g_hbm.at[idxB.at[3]], rows1, gs1)

        pltpu.make_async_copy(g_hbm.at[idxB.at[3]], rows1, gs1).wait()
        pltpu.async_copy(rows1, acc.at[idxB.at[7]], ss1, add=True)
        pltpu.make_async_copy(rows0, acc.at[idxB.at[6]], ss0).wait()
        pltpu.make_async_copy(idx_hbm.at[pl.ds(offA, 8)], idxA, semA).wait()
        pltpu.async_copy(g_hbm.at[idxA.at[0]], rows0, gs0)
        return 0

    lax.fori_loop(0, G // 2, two_groups, 0)
    pltpu.make_async_copy(rows1, acc.at[idxB.at[7]], ss1).wait()
    pltpu.make_async_copy(g_hbm.at[idxA.at[0]], rows0, gs0).wait()
    plsc.subcore_barrier()
    _copy_out(acc, out_hbm, c, s)


@functools.cache
def _sc_kernels():
    mesh = plsc.VectorSubcoreMesh(
        core_axis_name="c", subcore_axis_name="s", num_cores=NC, num_subcores=NS
    )
    sc_deg = pl.kernel(
        _sc_deg_body,
        out_type=jax.ShapeDtypeStruct((NC * NROW, D), jnp.float32),
        mesh=mesh,
        scratch_types=[
            pltpu.VMEM((K, CH), jnp.int32),
            pltpu.VMEM((CH, D), jnp.float32),
            pltpu.VMEM((ZR, D), jnp.float32),
            pltpu.VMEM_SHARED((NROW, D), jnp.float32),
            pltpu.SemaphoreType.DMA,
        ],
    )
    sc_agg = pl.kernel(
        _sc_agg_body,
        out_type=jax.ShapeDtypeStruct((NC * NROW, D), jnp.float32),
        mesh=mesh,
        scratch_types=[
            pltpu.VMEM((8, CH), jnp.int32),
            pltpu.VMEM((8, CH), jnp.int32),
            pltpu.VMEM((CH, D), jnp.float32),
            pltpu.VMEM((CH, D), jnp.float32),
            pltpu.VMEM_SHARED((NROW, D), jnp.float32),
            pltpu.SemaphoreType.DMA,
            pltpu.SemaphoreType.DMA,
            pltpu.SemaphoreType.DMA,
            pltpu.SemaphoreType.DMA,
            pltpu.SemaphoreType.DMA,
            pltpu.SemaphoreType.DMA,
        ],
    )
    return sc_deg, sc_agg


def _dinv(da_ref, db_ref):
    return lax.rsqrt(1.0 + da_ref[:, 0:1] + db_ref[:, 0:1])


def _mm1_body(x_ref, w_ref, da_ref, db_ref, o_ref):
    h = jnp.dot(x_ref[...], w_ref[...], preferred_element_type=jnp.float32)
    o_ref[...] = _dinv(da_ref, db_ref) * h


def _mm2_body(p0_ref, p1_ref, g_ref, da_ref, db_ref, b_ref, w_ref, o_ref):
    dinv = _dinv(da_ref, db_ref)
    t = dinv * (p0_ref[...] + p1_ref[...] + g_ref[...]) + b_ref[...]
    h = jnp.dot(t, w_ref[...], preferred_element_type=jnp.float32)
    o_ref[...] = dinv * h


def _fin_body(p0_ref, p1_ref, g_ref, da_ref, db_ref, b_ref, o_ref):
    dinv = _dinv(da_ref, db_ref)
    o_ref[...] = dinv * (p0_ref[...] + p1_ref[...] + g_ref[...]) + b_ref[...]


_row = pl.BlockSpec((R, D), lambda i: (i, 0))
_wspec = pl.BlockSpec((D, D), lambda i: (0, 0))
_bspec = pl.BlockSpec((1, D), lambda i: (0, 0))
_oshape = jax.ShapeDtypeStruct((N, D), jnp.float32)

_mm1 = pl.pallas_call(
    _mm1_body, grid=(GRID,),
    in_specs=[_row, _wspec, _row, _row],
    out_specs=_row, out_shape=_oshape,
)
_mm2 = pl.pallas_call(
    _mm2_body, grid=(GRID,),
    in_specs=[_row, _row, _row, _row, _row, _bspec, _wspec],
    out_specs=_row, out_shape=_oshape,
)
_fin = pl.pallas_call(
    _fin_body, grid=(GRID,),
    in_specs=[_row, _row, _row, _row, _row, _bspec],
    out_specs=_row, out_shape=_oshape,
)


def _halves(a):
    return a[:N], a[NROW:NROW + N]


def kernel(x, edge_index, W1, b1, W2, b2):
    src = edge_index[0]
    dst = edge_index[1]
    pad_idx = jnp.arange(PAD, dtype=jnp.int32)
    pad_src = pad_idx % N
    pad_dst = DUMP + pad_idx % (NROW - N)
    srcp = jnp.concatenate(
        [src.astype(jnp.int32).reshape(NW, E // NW),
         pad_src.reshape(NW, PAD // NW)], axis=1,
    ).reshape(NW * K, CH)
    dstp = jnp.concatenate(
        [dst.astype(jnp.int32).reshape(NW, E // NW),
         pad_dst.reshape(NW, PAD // NW)], axis=1,
    ).reshape(NW * K, CH)
    idxp = jnp.concatenate(
        [srcp.reshape(NW, G, 4, CH), dstp.reshape(NW, G, 4, CH)], axis=2
    ).reshape(NW * G * 8, CH)

    sc_deg, sc_agg = _sc_kernels()
    d0, d1 = _halves(sc_deg(dstp))
    g1 = _mm1(x, W1, d0, d1)
    p0, p1 = _halves(sc_agg(g1, idxp))
    g2 = _mm2(p0, p1, g1, d0, d1, b1.reshape(1, D), W2)
    q0, q1 = _halves(sc_agg(g2, idxp))
    return _fin(q0, q1, g2, d0, d1, b2.reshape(1, D))


# speedup vs baseline: 1.1112x; 1.1112x over previous
import functools

import jax
import jax.numpy as jnp
from jax import lax
from jax.experimental import pallas as pl
from jax.experimental.pallas import tpu as pltpu
from jax.experimental.pallas import tpu_sc as plsc

N = 10000
D = 128
E = 320000

NC = 2
NS = 16
NW = NC * NS

CH = 128
K = 80
G = K // 4
EW = K * CH
E_PAD = NW * EW
PAD = E_PAD - E
DEGW = 8

NROW = 10240
RPT = NROW // NS
DUMP = N
ZR = 80

R = 400
GRID = N // R


def _zero_acc(zb, acc, s, groups):
    def fill(i, _):
        zb[i // 8, pl.ds((i % 8) * 16, 16)] = jnp.zeros((16,), jnp.float32)
        return 0

    lax.fori_loop(0, ZR * 8, fill, 0)

    def zero(t, _):
        pltpu.sync_copy(zb, acc.at[pl.ds(s * RPT + t * ZR, ZR)])
        return 0

    lax.fori_loop(0, groups, zero, 0)


def _copy_out(acc, out_hbm, c, s):
    pltpu.sync_copy(
        acc.at[pl.ds(s * RPT, RPT)],
        out_hbm.at[pl.ds(c * NROW + s * RPT, RPT)],
    )


def _sc_deg_body(dst_hbm, out_hbm, dstv, ones, zb, acc, sem):
    c = lax.axis_index("c")
    s = lax.axis_index("s")
    wid = c * NS + s

    def fill(i, _):
        ones[i // 8, pl.ds((i % 8) * 16, 16)] = jnp.ones((16,), jnp.float32)
        return 0

    lax.fori_loop(0, CH * 8, fill, 0)
    _zero_acc(zb, acc, s, RPT // ZR)
    pltpu.sync_copy(dst_hbm.at[pl.ds(wid * K, K)], dstv)
    plsc.subcore_barrier()

    for j in range(DEGW):
        pltpu.async_copy(ones, acc.at[dstv.at[j]], sem, add=True)

    def chunk(j, _):
        pltpu.make_async_copy(ones, acc.at[dstv.at[j]], sem).wait()
        pltpu.async_copy(ones, acc.at[dstv.at[j + DEGW]], sem, add=True)
        return 0

    lax.fori_loop(0, K - DEGW, chunk, 0)
    for j in range(DEGW):
        pltpu.make_async_copy(ones, acc.at[dstv.at[0]], sem).wait()
    plsc.subcore_barrier()
    _copy_out(acc, out_hbm, c, s)


def _sc_agg_body(g_hbm, idx_hbm, out_hbm, idxA, idxB, rows0, rows1,
                 acc, gs0, gs1, semA, semB):
    c = lax.axis_index("c")
    s = lax.axis_index("s")
    wid = c * NS + s

    def fill(i, _):
        rows1[i // 8, pl.ds((i % 8) * 16, 16)] = jnp.zeros((16,), jnp.float32)
        return 0

    lax.fori_loop(0, CH * 8, fill, 0)

    def zero(t, _):
        pltpu.sync_copy(rows1, acc.at[pl.ds(s * RPT + t * CH, CH)])
        return 0

    lax.fori_loop(0, RPT // CH, zero, 0)

    pltpu.sync_copy(idx_hbm.at[pl.ds(wid * G * 8, 8)], idxA)
    pltpu.async_copy(idx_hbm.at[pl.ds((wid * G + 1) * 8, 8)], idxB, semB)
    plsc.subcore_barrier()

    pltpu.async_copy(g_hbm.at[idxA.at[0]], rows0, gs0)

    def two_groups(it, _):
        offA = (wid * G + jnp.minimum(2 * it + 2, G - 1)) * 8
        offB = (wid * G + jnp.minimum(2 * it + 3, G - 1)) * 8

        pltpu.async_copy(g_hbm.at[idxA.at[1]], rows1, gs1)
        pltpu.make_async_copy(g_hbm.at[idxA.at[0]], rows0, gs0).wait()
        pltpu.sync_copy(rows0, acc.at[idxA.at[4]], add=True)
        pltpu.async_copy(g_hbm.at[idxA.at[2]], rows0, gs0)
        pltpu.make_async_copy(g_hbm.at[idxA.at[1]], rows1, gs1).wait()
        pltpu.sync_copy(rows1, acc.at[idxA.at[5]], add=True)
        pltpu.async_copy(g_hbm.at[idxA.at[3]], rows1, gs1)
        pltpu.make_async_copy(g_hbm.at[idxA.at[2]], rows0, gs0).wait()
        pltpu.sync_copy(rows0, acc.at[idxA.at[6]], add=True)
        pltpu.make_async_copy(idx_hbm.at[pl.ds(offB, 8)], idxB, semB).wait()
        pltpu.async_copy(g_hbm.at[idxB.at[0]], rows0, gs0)
        pltpu.make_async_copy(g_hbm.at[idxA.at[3]], rows1, gs1).wait()
        pltpu.sync_copy(rows1, acc.at[idxA.at[7]], add=True)
        pltpu.async_copy(idx_hbm.at[pl.ds(offA, 8)], idxA, semA)

        pltpu.async_copy(g_hbm.at[idxB.at[1]], rows1, gs1)
        pltpu.make_async_copy(g_hbm.at[idxB.at[0]], rows0, gs0).wait()
        pltpu.sync_copy(rows0, acc.at[idxB.at[4]], add=True)
        pltpu.async_copy(g_hbm.at[idxB.at[2]], rows0, gs0)
        pltpu.make_async_copy(g_hbm.at[idxB.at[1]], rows1, gs1).wait()
        pltpu.sync_copy(rows1, acc.at[idxB.at[5]], add=True)
        pltpu.async_copy(g_hbm.at[idxB.at[3]], rows1, gs1)
        pltpu.make_async_copy(g_hbm.at[idxB.at[2]], rows0, gs0).wait()
        pltpu.sync_copy(rows0, acc.at[idxB.at[6]], add=True)
        pltpu.make_async_copy(idx_hbm.at[pl.ds(offA, 8)], idxA, semA).wait()
        pltpu.async_copy(g_hbm.at[idxA.at[0]], rows0, gs0)
        pltpu.make_async_copy(g_hbm.at[idxB.at[3]], rows1, gs1).wait()
        pltpu.sync_copy(rows1, acc.at[idxB.at[7]], add=True)
        pltpu.async_copy(idx_hbm.at[pl.ds(offB, 8)], idxB, semB)
        return 0

    lax.fori_loop(0, G // 2, two_groups, 0)
    pltpu.make_async_copy(g_hbm.at[idxA.at[0]], rows0, gs0).wait()
    pltpu.make_async_copy(idx_hbm.at[pl.ds(0, 8)], idxB, semB).wait()
    plsc.subcore_barrier()
    _copy_out(acc, out_hbm, c, s)


@functools.cache
def _sc_kernels():
    mesh = plsc.VectorSubcoreMesh(
        core_axis_name="c", subcore_axis_name="s", num_cores=NC, num_subcores=NS
    )
    sc_deg = pl.kernel(
        _sc_deg_body,
        out_type=jax.ShapeDtypeStruct((NC * NROW, D), jnp.float32),
        mesh=mesh,
        scratch_types=[
            pltpu.VMEM((K, CH), jnp.int32),
            pltpu.VMEM((CH, D), jnp.float32),
            pltpu.VMEM((ZR, D), jnp.float32),
            pltpu.VMEM_SHARED((NROW, D), jnp.float32),
            pltpu.SemaphoreType.DMA,
        ],
    )
    sc_agg = pl.kernel(
        _sc_agg_body,
        out_type=jax.ShapeDtypeStruct((NC * NROW, D), jnp.float32),
        mesh=mesh,
        scratch_types=[
            pltpu.VMEM((8, CH), jnp.int32),
            pltpu.VMEM((8, CH), jnp.int32),
            pltpu.VMEM((CH, D), jnp.float32),
            pltpu.VMEM((CH, D), jnp.float32),
            pltpu.VMEM_SHARED((NROW, D), jnp.float32),
            pltpu.SemaphoreType.DMA,
            pltpu.SemaphoreType.DMA,
            pltpu.SemaphoreType.DMA,
            pltpu.SemaphoreType.DMA,
        ],
    )
    return sc_deg, sc_agg


def _dinv(da_ref, db_ref):
    return lax.rsqrt(1.0 + da_ref[:, 0:1] + db_ref[:, 0:1])


def _mmraw_body(x_ref, w_ref, o_ref):
    o_ref[...] = jnp.dot(x_ref[...], w_ref[...],
                         preferred_element_type=jnp.float32)


def _scale_body(h_ref, da_ref, db_ref, o_ref):
    o_ref[...] = _dinv(da_ref, db_ref) * h_ref[...]


def _mm2_body(p0_ref, p1_ref, g_ref, da_ref, db_ref, b_ref, w_ref, o_ref):
    dinv = _dinv(da_ref, db_ref)
    t = dinv * (p0_ref[...] + p1_ref[...] + g_ref[...]) + b_ref[...]
    h = jnp.dot(t, w_ref[...], preferred_element_type=jnp.float32)
    o_ref[...] = dinv * h


def _fin_body(p0_ref, p1_ref, g_ref, da_ref, db_ref, b_ref, o_ref):
    dinv = _dinv(da_ref, db_ref)
    o_ref[...] = dinv * (p0_ref[...] + p1_ref[...] + g_ref[...]) + b_ref[...]


_row = pl.BlockSpec((R, D), lambda i: (i, 0))
_wspec = pl.BlockSpec((D, D), lambda i: (0, 0))
_bspec = pl.BlockSpec((1, D), lambda i: (0, 0))
_oshape = jax.ShapeDtypeStruct((N, D), jnp.float32)

_mmraw = pl.pallas_call(
    _mmraw_body, grid=(GRID,),
    in_specs=[_row, _wspec],
    out_specs=_row, out_shape=_oshape,
)
_scale = pl.pallas_call(
    _scale_body, grid=(GRID,),
    in_specs=[_row, _row, _row],
    out_specs=_row, out_shape=_oshape,
)
_mm2 = pl.pallas_call(
    _mm2_body, grid=(GRID,),
    in_specs=[_row, _row, _row, _row, _row, _bspec, _wspec],
    out_specs=_row, out_shape=_oshape,
)
_fin = pl.pallas_call(
    _fin_body, grid=(GRID,),
    in_specs=[_row, _row, _row, _row, _row, _bspec],
    out_specs=_row, out_shape=_oshape,
)


def _halves(a):
    return a[:N], a[NROW:NROW + N]


def kernel(x, edge_index, W1, b1, W2, b2):
    src = edge_index[0]
    dst = edge_index[1]
    pad_idx = jnp.arange(PAD, dtype=jnp.int32)
    pad_src = pad_idx % N
    pad_dst = DUMP + pad_idx % (NROW - N)
    srcp = jnp.concatenate(
        [src.astype(jnp.int32).reshape(NW, E // NW),
         pad_src.reshape(NW, PAD // NW)], axis=1,
    ).reshape(NW * K, CH)
    dstp = jnp.concatenate(
        [dst.astype(jnp.int32).reshape(NW, E // NW),
         pad_dst.reshape(NW, PAD // NW)], axis=1,
    ).reshape(NW * K, CH)
    idxp = jnp.concatenate(
        [srcp.reshape(NW, G, 4, CH), dstp.reshape(NW, G, 4, CH)], axis=2
    ).reshape(NW * G * 8, CH)

    sc_deg, sc_agg = _sc_kernels()
    h1 = _mmraw(x, W1)
    d0, d1 = _halves(sc_deg(dstp))
    g1 = _scale(h1, d0, d1)
    p0, p1 = _halves(sc_agg(g1, idxp))
    g2 = _mm2(p0, p1, g1, d0, d1, b1.reshape(1, D), W2)
    q0, q1 = _halves(sc_agg(g2, idxp))
    return _fin(q0, q1, g2, d0, d1, b2.reshape(1, D))


# speedup vs baseline: 1.1679x; 1.0511x over previous
import functools

import jax
import jax.numpy as jnp
from jax import lax
from jax.experimental import pallas as pl
from jax.experimental.pallas import tpu as pltpu
from jax.experimental.pallas import tpu_sc as plsc

N = 10000
D = 128
E = 320000

NC = 2
NS = 16
NW = NC * NS

CH = 128
K = 80
G = K // 4
EW = K * CH
E_PAD = NW * EW
PAD = E_PAD - E
DEGW = 8

NROW = 10240
RPT = NROW // NS
DUMP = N
ZR = 80

R = 400
GRID = N // R
OFFC = 10400
POFF = OFFC // R


def _zero_acc(zb, acc, s, groups):
    def fill(i, _):
        zb[i // 8, pl.ds((i % 8) * 16, 16)] = jnp.zeros((16,), jnp.float32)
        return 0

    lax.fori_loop(0, ZR * 8, fill, 0)

    def zero(t, _):
        pltpu.sync_copy(zb, acc.at[pl.ds(s * RPT + t * ZR, ZR)])
        return 0

    lax.fori_loop(0, groups, zero, 0)


def _copy_out(acc, out_hbm, c, s):
    pltpu.sync_copy(
        acc.at[pl.ds(s * RPT, RPT)],
        out_hbm.at[pl.ds(c * OFFC + s * RPT, RPT)],
    )


def _sc_deg_body(dst_hbm, out_hbm, dstv, ones, zb, acc, sem):
    c = lax.axis_index("c")
    s = lax.axis_index("s")
    wid = c * NS + s

    def fill(i, _):
        ones[i // 8, pl.ds((i % 8) * 16, 16)] = jnp.ones((16,), jnp.float32)
        return 0

    lax.fori_loop(0, CH * 8, fill, 0)
    _zero_acc(zb, acc, s, RPT // ZR)
    pltpu.sync_copy(dst_hbm.at[pl.ds(wid * K, K)], dstv)
    plsc.subcore_barrier()

    for j in range(DEGW):
        pltpu.async_copy(ones, acc.at[dstv.at[j]], sem, add=True)

    def chunk(j, _):
        pltpu.make_async_copy(ones, acc.at[dstv.at[j]], sem).wait()
        pltpu.async_copy(ones, acc.at[dstv.at[j + DEGW]], sem, add=True)
        return 0

    lax.fori_loop(0, K - DEGW, chunk, 0)
    for j in range(DEGW):
        pltpu.make_async_copy(ones, acc.at[dstv.at[0]], sem).wait()
    plsc.subcore_barrier()
    _copy_out(acc, out_hbm, c, s)


def _sc_agg_body(g_hbm, idx_hbm, out_hbm, idxA, idxB, rows0, rows1,
                 acc, gs0, gs1, semA, semB):
    c = lax.axis_index("c")
    s = lax.axis_index("s")
    wid = c * NS + s

    def fill(i, _):
        rows1[i // 8, pl.ds((i % 8) * 16, 16)] = jnp.zeros((16,), jnp.float32)
        return 0

    lax.fori_loop(0, CH * 8, fill, 0)

    def zero(t, _):
        pltpu.sync_copy(rows1, acc.at[pl.ds(s * RPT + t * CH, CH)])
        return 0

    lax.fori_loop(0, RPT // CH, zero, 0)

    pltpu.sync_copy(idx_hbm.at[pl.ds(wid * G * 8, 8)], idxA)
    pltpu.async_copy(idx_hbm.at[pl.ds((wid * G + 1) * 8, 8)], idxB, semB)
    plsc.subcore_barrier()

    pltpu.async_copy(g_hbm.at[idxA.at[0]], rows0, gs0)

    def two_groups(it, _):
        offA = (wid * G + jnp.minimum(2 * it + 2, G - 1)) * 8
        offB = (wid * G + jnp.minimum(2 * it + 3, G - 1)) * 8

        pltpu.async_copy(g_hbm.at[idxA.at[1]], rows1, gs1)
        pltpu.make_async_copy(g_hbm.at[idxA.at[0]], rows0, gs0).wait()
        pltpu.sync_copy(rows0, acc.at[idxA.at[4]], add=True)
        pltpu.async_copy(g_hbm.at[idxA.at[2]], rows0, gs0)
        pltpu.make_async_copy(g_hbm.at[idxA.at[1]], rows1, gs1).wait()
        pltpu.sync_copy(rows1, acc.at[idxA.at[5]], add=True)
        pltpu.async_copy(g_hbm.at[idxA.at[3]], rows1, gs1)
        pltpu.make_async_copy(g_hbm.at[idxA.at[2]], rows0, gs0).wait()
        pltpu.sync_copy(rows0, acc.at[idxA.at[6]], add=True)
        pltpu.make_async_copy(idx_hbm.at[pl.ds(offB, 8)], idxB, semB).wait()
        pltpu.async_copy(g_hbm.at[idxB.at[0]], rows0, gs0)
        pltpu.make_async_copy(g_hbm.at[idxA.at[3]], rows1, gs1).wait()
        pltpu.sync_copy(rows1, acc.at[idxA.at[7]], add=True)
        pltpu.async_copy(idx_hbm.at[pl.ds(offA, 8)], idxA, semA)

        pltpu.async_copy(g_hbm.at[idxB.at[1]], rows1, gs1)
        pltpu.make_async_copy(g_hbm.at[idxB.at[0]], rows0, gs0).wait()
        pltpu.sync_copy(rows0, acc.at[idxB.at[4]], add=True)
        pltpu.async_copy(g_hbm.at[idxB.at[2]], rows0, gs0)
        pltpu.make_async_copy(g_hbm.at[idxB.at[1]], rows1, gs1).wait()
        pltpu.sync_copy(rows1, acc.at[idxB.at[5]], add=True)
        pltpu.async_copy(g_hbm.at[idxB.at[3]], rows1, gs1)
        pltpu.make_async_copy(g_hbm.at[idxB.at[2]], rows0, gs0).wait()
        pltpu.sync_copy(rows0, acc.at[idxB.at[6]], add=True)
        pltpu.make_async_copy(idx_hbm.at[pl.ds(offA, 8)], idxA, semA).wait()
        pltpu.async_copy(g_hbm.at[idxA.at[0]], rows0, gs0)
        pltpu.make_async_copy(g_hbm.at[idxB.at[3]], rows1, gs1).wait()
        pltpu.sync_copy(rows1, acc.at[idxB.at[7]], add=True)
        pltpu.async_copy(idx_hbm.at[pl.ds(offB, 8)], idxB, semB)
        return 0

    lax.fori_loop(0, G // 2, two_groups, 0)
    pltpu.make_async_copy(g_hbm.at[idxA.at[0]], rows0, gs0).wait()
    pltpu.make_async_copy(idx_hbm.at[pl.ds(0, 8)], idxB, semB).wait()
    plsc.subcore_barrier()
    _copy_out(acc, out_hbm, c, s)


@functools.cache
def _sc_kernels():
    mesh = plsc.VectorSubcoreMesh(
        core_axis_name="c", subcore_axis_name="s", num_cores=NC, num_subcores=NS
    )
    sc_deg = pl.kernel(
        _sc_deg_body,
        out_type=jax.ShapeDtypeStruct((OFFC + NROW, D), jnp.float32),
        mesh=mesh,
        scratch_types=[
            pltpu.VMEM((K, CH), jnp.int32),
            pltpu.VMEM((CH, D), jnp.float32),
            pltpu.VMEM((ZR, D), jnp.float32),
            pltpu.VMEM_SHARED((NROW, D), jnp.float32),
            pltpu.SemaphoreType.DMA,
        ],
    )
    sc_agg = pl.kernel(
        _sc_agg_body,
        out_type=jax.ShapeDtypeStruct((OFFC + NROW, D), jnp.float32),
        mesh=mesh,
        scratch_types=[
            pltpu.VMEM((8, CH), jnp.int32),
            pltpu.VMEM((8, CH), jnp.int32),
            pltpu.VMEM((CH, D), jnp.float32),
            pltpu.VMEM((CH, D), jnp.float32),
            pltpu.VMEM_SHARED((NROW, D), jnp.float32),
            pltpu.SemaphoreType.DMA,
            pltpu.SemaphoreType.DMA,
            pltpu.SemaphoreType.DMA,
            pltpu.SemaphoreType.DMA,
        ],
    )
    return sc_deg, sc_agg


def _dinv(da_ref, db_ref):
    return lax.rsqrt(1.0 + da_ref[:, 0:1] + db_ref[:, 0:1])


def _mmraw_body(x_ref, w_ref, o_ref):
    o_ref[...] = jnp.dot(x_ref[...], w_ref[...],
                         preferred_element_type=jnp.float32)


def _scale_body(h_ref, da_ref, db_ref, o_ref):
    o_ref[...] = _dinv(da_ref, db_ref) * h_ref[...]


def _mm2_body(p0_ref, p1_ref, g_ref, da_ref, db_ref, b_ref, w_ref, o_ref):
    dinv = _dinv(da_ref, db_ref)
    t = dinv * (p0_ref[...] + p1_ref[...] + g_ref[...]) + b_ref[...]
    h = jnp.dot(t, w_ref[...], preferred_element_type=jnp.float32)
    o_ref[...] = dinv * h


def _fin_body(p0_ref, p1_ref, g_ref, da_ref, db_ref, b_ref, o_ref):
    dinv = _dinv(da_ref, db_ref)
    o_ref[...] = dinv * (p0_ref[...] + p1_ref[...] + g_ref[...]) + b_ref[...]


_row = pl.BlockSpec((R, D), lambda i: (i, 0))
_row1 = pl.BlockSpec((R, D), lambda i: (i + POFF, 0))
_wspec = pl.BlockSpec((D, D), lambda i: (0, 0))
_bspec = pl.BlockSpec((1, D), lambda i: (0, 0))
_oshape = jax.ShapeDtypeStruct((N, D), jnp.float32)

_mmraw = pl.pallas_call(
    _mmraw_body, grid=(GRID,),
    in_specs=[_row, _wspec],
    out_specs=_row, out_shape=_oshape,
)
_scale = pl.pallas_call(
    _scale_body, grid=(GRID,),
    in_specs=[_row, _row, _row1],
    out_specs=_row, out_shape=_oshape,
)
_mm2 = pl.pallas_call(
    _mm2_body, grid=(GRID,),
    in_specs=[_row, _row1, _row, _row, _row1, _bspec, _wspec],
    out_specs=_row, out_shape=_oshape,
)
_fin = pl.pallas_call(
    _fin_body, grid=(GRID,),
    in_specs=[_row, _row1, _row, _row, _row1, _bspec],
    out_specs=_row, out_shape=_oshape,
)


def kernel(x, edge_index, W1, b1, W2, b2):
    src = edge_index[0]
    dst = edge_index[1]
    pad_idx = jnp.arange(PAD, dtype=jnp.int32)
    pad_src = pad_idx % N
    pad_dst = DUMP + pad_idx % (NROW - N)
    srcp = jnp.concatenate(
        [src.astype(jnp.int32).reshape(NW, E // NW),
         pad_src.reshape(NW, PAD // NW)], axis=1,
    ).reshape(NW * K, CH)
    dstp = jnp.concatenate(
        [dst.astype(jnp.int32).reshape(NW, E // NW),
         pad_dst.reshape(NW, PAD // NW)], axis=1,
    ).reshape(NW * K, CH)
    idxp = jnp.concatenate(
        [srcp.reshape(NW, G, 4, CH), dstp.reshape(NW, G, 4, CH)], axis=2
    ).reshape(NW * G * 8, CH)

    sc_deg, sc_agg = _sc_kernels()
    h1 = _mmraw(x, W1)
    degp = sc_deg(dstp)
    g1 = _scale(h1, degp, degp)
    s1 = sc_agg(g1, idxp)
    g2 = _mm2(s1, s1, g1, degp, degp, b1.reshape(1, D), W2)
    s2 = sc_agg(g2, idxp)
    return _fin(s2, s2, g2, degp, degp, b2.reshape(1, D))
